# baseline (speedup 1.0000x reference)
import functools

import jax
import jax.numpy as jnp
from jax import lax
from jax.experimental import pallas as pl
from jax.experimental.pallas import tpu as pltpu
from jax.experimental.pallas import tpu_sc as plsc

_NUM_CORES = 2
_NUM_SUBCORES = 16
_NUM_WORKERS = _NUM_CORES * _NUM_SUBCORES
_LANES = 16


def _pick_chunk(per_worker: int) -> int:
    for c in range(min(4000, per_worker), 7, -8):
        if per_worker % c == 0:
            return c
    return per_worker


def _sc_body(px_hbm, py_hbm, pz_hbm, ei_hbm, ew_hbm, out_hbm,
             idxr_v, idxc_v, ew_v, out_v,
             xr_v, yr_v, zr_v, xc_v, yc_v, zc_v, sem_r, sem_c,
             *, per_worker: int, chunk: int, n_edges: int):
    wid = lax.axis_index("s") * _NUM_CORES + lax.axis_index("c")
    base = wid * per_worker
    n_chunks = per_worker // chunk
    n_vec = chunk // _LANES

    lane_iota = lax.iota(jnp.int32, _LANES)
    one = jnp.full((_LANES,), 1, jnp.int32)
    two = jnp.full((_LANES,), 2, jnp.int32)
    three = jnp.full((_LANES,), 3, jnp.int32)
    half = jnp.float32(0.5)
    threehalf = jnp.float32(1.5)
    magic = jnp.int32(0x5F3759DF)

    def chunk_body(k, _):
        off = base + k * chunk
        pltpu.sync_copy(ei_hbm.at[pl.ds(off, chunk)], idxr_v)
        pltpu.sync_copy(ei_hbm.at[pl.ds(n_edges + off, chunk)], idxc_v)
        pltpu.sync_copy(ew_hbm.at[pl.ds(off, chunk)], ew_v)
        cps = [
            pltpu.async_copy(px_hbm.at[idxr_v], xr_v, sem_r),
            pltpu.async_copy(py_hbm.at[idxr_v], yr_v, sem_r),
            pltpu.async_copy(pz_hbm.at[idxr_v], zr_v, sem_r),
            pltpu.async_copy(px_hbm.at[idxc_v], xc_v, sem_c),
            pltpu.async_copy(py_hbm.at[idxc_v], yc_v, sem_c),
            pltpu.async_copy(pz_hbm.at[idxc_v], zc_v, sem_c),
        ]
        for cp in cps:
            cp.wait()

        def vec_body(i, _):
            e0 = i * _LANES
            f0 = (lane_iota + e0) * 4
            f1 = f0 + one
            f2 = f0 + two
            f3 = f0 + three
            rx = xr_v[pl.ds(e0, _LANES)]
            ry = yr_v[pl.ds(e0, _LANES)]
            rz = zr_v[pl.ds(e0, _LANES)]
            cx = xc_v[pl.ds(e0, _LANES)]
            cy = yc_v[pl.ds(e0, _LANES)]
            cz = zc_v[pl.ds(e0, _LANES)]
            dx = cx - rx
            dy = cy - ry
            dz = cz - rz
            s = dx * dx + dy * dy + dz * dz
            s_bits = lax.bitcast_convert_type(s, jnp.int32)
            y = lax.bitcast_convert_type(magic - (s_bits >> 1), jnp.float32)
            xh = s * half
            y = y * (threehalf - xh * y * y)
            y = y * (threehalf - xh * y * y)
            y = y * (threehalf - xh * y * y)
            h = y * half
            plsc.store_scatter(out_v, [f0], dx * h + half)
            plsc.store_scatter(out_v, [f1], dy * h + half)
            plsc.store_scatter(out_v, [f2], dz * h + half)
            plsc.store_scatter(out_v, [f3], ew_v[pl.ds(e0, _LANES)])
            return _

        lax.fori_loop(0, n_vec, vec_body, None)
        pltpu.sync_copy(out_v, out_hbm.at[pl.ds(off * 4, chunk * 4)])
        return _

    lax.fori_loop(0, n_chunks, chunk_body, None)


@functools.cache
def _build(n_edges: int):
    per_worker = n_edges // _NUM_WORKERS
    chunk = _pick_chunk(per_worker)
    mesh = plsc.VectorSubcoreMesh(core_axis_name="c", subcore_axis_name="s",
                                  num_cores=_NUM_CORES,
                                  num_subcores=_NUM_SUBCORES)
    return pl.kernel(
        functools.partial(_sc_body, per_worker=per_worker, chunk=chunk,
                          n_edges=n_edges),
        out_type=jax.ShapeDtypeStruct((n_edges * 4,), jnp.float32),
        mesh=mesh,
        scratch_types=[
            pltpu.VMEM((chunk,), jnp.int32),
            pltpu.VMEM((chunk,), jnp.int32),
            pltpu.VMEM((chunk,), jnp.float32),
            pltpu.VMEM((chunk * 4,), jnp.float32),
            pltpu.VMEM((chunk,), jnp.float32),
            pltpu.VMEM((chunk,), jnp.float32),
            pltpu.VMEM((chunk,), jnp.float32),
            pltpu.VMEM((chunk,), jnp.float32),
            pltpu.VMEM((chunk,), jnp.float32),
            pltpu.VMEM((chunk,), jnp.float32),
            pltpu.SemaphoreType.DMA,
            pltpu.SemaphoreType.DMA,
        ],
        compiler_params=pltpu.CompilerParams(needs_layout_passes=False),
    )


def kernel(pos, edge_index, edge_weight):
    n_edges = edge_weight.shape[0]
    posf = pos.astype(jnp.float32)
    px, py, pz = posf[:, 0], posf[:, 1], posf[:, 2]
    ei_flat = edge_index.astype(jnp.int32).reshape(2 * n_edges)
    flat = _build(n_edges)(px, py, pz, ei_flat,
                           edge_weight.astype(jnp.float32))
    return flat.reshape(n_edges, 4)


# speedup vs baseline: 2.9873x; 2.9873x over previous
import functools

import jax
import jax.numpy as jnp
from jax import lax
from jax.experimental import pallas as pl
from jax.experimental.pallas import tpu as pltpu
from jax.experimental.pallas import tpu_sc as plsc

_NUM_CORES = 2
_NUM_SUBCORES = 16
_NUM_WORKERS = _NUM_CORES * _NUM_SUBCORES
_LANES = 16
_BLK = 128


def _pick_chunk(n_edges: int) -> int:
    for c in range(3200, _BLK - 1, -_BLK):
        if n_edges % c == 0:
            return c
    return _BLK


def _sc_body(px_hbm, py_hbm, pz_hbm, ei_hbm,
             ox_hbm, oy_hbm, oz_hbm,
             idx_v, xg_v, yg_v, zg_v, ox_v, oy_v, oz_v, sem,
             *, chunk: int, n_chunks: int):
    wid = lax.axis_index("s") * _NUM_CORES + lax.axis_index("c")
    n_mine = (n_chunks - wid + _NUM_WORKERS - 1) // _NUM_WORKERS
    n_vec = chunk // _LANES
    grp_per_blk = _BLK // _LANES

    half = jnp.float32(0.5)
    threehalf = jnp.float32(1.5)
    magic = jnp.int32(0x5F3759DF)

    def chunk_body(j, _):
        k = wid + j * _NUM_WORKERS
        off = k * chunk
        pltpu.sync_copy(ei_hbm.at[pl.ds(off * 2, chunk * 2)], idx_v)
        cps = [
            pltpu.async_copy(px_hbm.at[idx_v], xg_v, sem),
            pltpu.async_copy(py_hbm.at[idx_v], yg_v, sem),
            pltpu.async_copy(pz_hbm.at[idx_v], zg_v, sem),
        ]
        for cp in cps:
            cp.wait()

        def vec_body(i, _):
            e0 = i * _LANES
            g0 = e0 + (i // grp_per_blk) * _BLK
            rsl = pl.ds(g0, _LANES)
            csl = pl.ds(g0 + _BLK, _LANES)
            osl = pl.ds(e0, _LANES)
            dx = xg_v[csl] - xg_v[rsl]
            dy = yg_v[csl] - yg_v[rsl]
            dz = zg_v[csl] - zg_v[rsl]
            s = dx * dx + dy * dy + dz * dz
            s_bits = lax.bitcast_convert_type(s, jnp.int32)
            y = lax.bitcast_convert_type(magic - (s_bits >> 1), jnp.float32)
            xh = s * half
            y = y * (threehalf - xh * y * y)
            y = y * (threehalf - xh * y * y)
            y = y * (threehalf - xh * y * y)
            h = y * half
            ox_v[osl] = dx * h + half
            oy_v[osl] = dy * h + half
            oz_v[osl] = dz * h + half
            return _

        lax.fori_loop(0, n_vec, vec_body, None)
        pltpu.sync_copy(ox_v, ox_hbm.at[pl.ds(off, chunk)])
        pltpu.sync_copy(oy_v, oy_hbm.at[pl.ds(off, chunk)])
        pltpu.sync_copy(oz_v, oz_hbm.at[pl.ds(off, chunk)])
        return _

    lax.fori_loop(0, n_mine, chunk_body, None)


@functools.cache
def _build(n_edges: int):
    chunk = _pick_chunk(n_edges)
    n_chunks = n_edges // chunk
    mesh = plsc.VectorSubcoreMesh(core_axis_name="c", subcore_axis_name="s",
                                  num_cores=_NUM_CORES,
                                  num_subcores=_NUM_SUBCORES)
    comp = jax.ShapeDtypeStruct((n_edges,), jnp.float32)
    return pl.kernel(
        functools.partial(_sc_body, chunk=chunk, n_chunks=n_chunks),
        out_type=(comp, comp, comp),
        mesh=mesh,
        scratch_types=[
            pltpu.VMEM((chunk * 2,), jnp.int32),
            pltpu.VMEM((chunk * 2,), jnp.float32),
            pltpu.VMEM((chunk * 2,), jnp.float32),
            pltpu.VMEM((chunk * 2,), jnp.float32),
            pltpu.VMEM((chunk,), jnp.float32),
            pltpu.VMEM((chunk,), jnp.float32),
            pltpu.VMEM((chunk,), jnp.float32),
            pltpu.SemaphoreType.DMA,
        ],
        compiler_params=pltpu.CompilerParams(needs_layout_passes=False),
    )


def kernel(pos, edge_index, edge_weight):
    n_edges = edge_weight.shape[0]
    posf = pos.astype(jnp.float32)
    px, py, pz = posf[:, 0], posf[:, 1], posf[:, 2]
    ei_blk = (edge_index.astype(jnp.int32)
              .reshape(2, n_edges // _BLK, _BLK)
              .transpose(1, 0, 2)
              .reshape(2 * n_edges))
    ox, oy, oz = _build(n_edges)(px, py, pz, ei_blk)
    return jnp.stack([ox, oy, oz, edge_weight.astype(jnp.float32)], axis=1)


# speedup vs baseline: 5.0174x; 1.6796x over previous
import functools

import jax
import jax.numpy as jnp
from jax import lax
from jax.experimental import pallas as pl
from jax.experimental.pallas import tpu as pltpu
from jax.experimental.pallas import tpu_sc as plsc

_NUM_CORES = 2
_NUM_SUBCORES = 16
_NUM_WORKERS = _NUM_CORES * _NUM_SUBCORES
_LANES = 16
_BLK = 128


def _pick_chunk(n_edges: int) -> int:
    for c in range(3200, _BLK - 1, -_BLK):
        if n_edges % c == 0:
            return c
    return _BLK


def _sc_body(pos4_hbm, ei_hbm, ox_hbm, oy_hbm, oz_hbm,
             idx_v, rows_v, ox_v, oy_v, oz_v, sem,
             *, chunk: int, n_chunks: int):
    wid = lax.axis_index("s") * _NUM_CORES + lax.axis_index("c")
    n_mine = (n_chunks - wid + _NUM_WORKERS - 1) // _NUM_WORKERS
    n_vec = chunk // _LANES
    grp_per_blk = _BLK // _LANES

    lane_iota = lax.iota(jnp.int32, _LANES)
    comp = [jnp.full((_LANES,), j, jnp.int32) for j in range(3)]
    half = jnp.float32(0.5)
    threehalf = jnp.float32(1.5)
    magic = jnp.int32(0x5F3759DF)

    def chunk_body(j, _):
        k = wid + j * _NUM_WORKERS
        off = k * chunk
        pltpu.sync_copy(ei_hbm.at[pl.ds(off * 2, chunk * 2)], idx_v)
        pltpu.async_copy(pos4_hbm.at[idx_v], rows_v, sem).wait()

        def vec_body(i, _):
            e0 = i * _LANES
            g0 = e0 + (i // grp_per_blk) * _BLK
            eid_r = lane_iota + g0
            eid_c = eid_r + _BLK
            rx = plsc.load_gather(rows_v, [eid_r, comp[0]])
            ry = plsc.load_gather(rows_v, [eid_r, comp[1]])
            rz = plsc.load_gather(rows_v, [eid_r, comp[2]])
            cx = plsc.load_gather(rows_v, [eid_c, comp[0]])
            cy = plsc.load_gather(rows_v, [eid_c, comp[1]])
            cz = plsc.load_gather(rows_v, [eid_c, comp[2]])
            osl = pl.ds(e0, _LANES)
            dx = cx - rx
            dy = cy - ry
            dz = cz - rz
            s = dx * dx + dy * dy + dz * dz
            s_bits = lax.bitcast_convert_type(s, jnp.int32)
            y = lax.bitcast_convert_type(magic - (s_bits >> 1), jnp.float32)
            xh = s * half
            y = y * (threehalf - xh * y * y)
            y = y * (threehalf - xh * y * y)
            y = y * (threehalf - xh * y * y)
            h = y * half
            ox_v[osl] = dx * h + half
            oy_v[osl] = dy * h + half
            oz_v[osl] = dz * h + half
            return _

        lax.fori_loop(0, n_vec, vec_body, None)
        pltpu.sync_copy(ox_v, ox_hbm.at[pl.ds(off, chunk)])
        pltpu.sync_copy(oy_v, oy_hbm.at[pl.ds(off, chunk)])
        pltpu.sync_copy(oz_v, oz_hbm.at[pl.ds(off, chunk)])
        return _

    lax.fori_loop(0, n_mine, chunk_body, None)


@functools.cache
def _build(n_edges: int):
    chunk = _pick_chunk(n_edges)
    n_chunks = n_edges // chunk
    mesh = plsc.VectorSubcoreMesh(core_axis_name="c", subcore_axis_name="s",
                                  num_cores=_NUM_CORES,
                                  num_subcores=_NUM_SUBCORES)
    comp = jax.ShapeDtypeStruct((n_edges,), jnp.float32)
    return pl.kernel(
        functools.partial(_sc_body, chunk=chunk, n_chunks=n_chunks),
        out_type=(comp, comp, comp),
        mesh=mesh,
        scratch_types=[
            pltpu.VMEM((chunk * 2,), jnp.int32),
            pltpu.VMEM((chunk * 2, 8), jnp.float32),
            pltpu.VMEM((chunk,), jnp.float32),
            pltpu.VMEM((chunk,), jnp.float32),
            pltpu.VMEM((chunk,), jnp.float32),
            pltpu.SemaphoreType.DMA,
        ],
        compiler_params=pltpu.CompilerParams(needs_layout_passes=False,
                                             use_tc_tiling_on_sc=False),
    )


def kernel(pos, edge_index, edge_weight):
    n_edges = edge_weight.shape[0]
    pos4 = jnp.concatenate(
        [pos.astype(jnp.float32),
         jnp.zeros((pos.shape[0], 5), jnp.float32)], axis=1)
    ei_blk = (edge_index.astype(jnp.int32)
              .reshape(2, n_edges // _BLK, _BLK)
              .transpose(1, 0, 2)
              .reshape(2 * n_edges))
    ox, oy, oz = _build(n_edges)(pos4, ei_blk)
    return jnp.stack([ox, oy, oz, edge_weight.astype(jnp.float32)], axis=1)


# speedup vs baseline: 8.0335x; 1.6011x over previous
import functools

import jax
import jax.numpy as jnp
from jax import lax
from jax.experimental import pallas as pl
from jax.experimental.pallas import tpu as pltpu
from jax.experimental.pallas import tpu_sc as plsc

_NUM_CORES = 2
_NUM_SUBCORES = 16
_NUM_WORKERS = _NUM_CORES * _NUM_SUBCORES
_LANES = 16
_BLK = 128


def _pick_chunk(n_edges: int) -> int:
    for c in range(2560, _BLK - 1, -_BLK):
        if n_edges % c == 0:
            return c
    return _BLK


def _sc_body(pos4_hbm, ei_hbm, ox_hbm, oy_hbm, oz_hbm,
             idx_v, rows_v, ox_v, oy_v, oz_v, sem,
             *, chunk: int, n_chunks: int):
    wid = lax.axis_index("s") * _NUM_CORES + lax.axis_index("c")
    n_mine = (n_chunks - wid + _NUM_WORKERS - 1) // _NUM_WORKERS
    n_vec = chunk // _LANES
    grp_per_blk = _BLK // _LANES

    lane_iota = lax.iota(jnp.int32, _LANES)
    comp = [jnp.full((_LANES,), j, jnp.int32) for j in range(3)]
    half = jnp.float32(0.5)
    threehalf = jnp.float32(1.5)
    magic = jnp.int32(0x5F3759DF)

    def start_fetch(j):
        sel = (j % 2) * chunk * 2
        off = (wid + j * _NUM_WORKERS) * chunk
        idx_half = idx_v.at[pl.ds(sel, chunk * 2)]
        pltpu.sync_copy(ei_hbm.at[pl.ds(off * 2, chunk * 2)], idx_half)
        pltpu.async_copy(pos4_hbm.at[idx_half],
                         rows_v.at[pl.ds(sel, chunk * 2), :],
                         sem.at[j % 2])

    start_fetch(0)

    def chunk_body(j, _):
        k = wid + j * _NUM_WORKERS
        off = k * chunk
        sel = (j % 2) * chunk * 2

        @pl.when(j + 1 < n_mine)
        def _prefetch():
            start_fetch(j + 1)

        pltpu.make_async_copy(pos4_hbm.at[idx_v.at[pl.ds(sel, chunk * 2)]],
                              rows_v.at[pl.ds(sel, chunk * 2), :],
                              sem.at[j % 2]).wait()

        def vec_body(i, _):
            e0 = i * _LANES
            g0 = e0 + (i // grp_per_blk) * _BLK
            eid_r = lane_iota + sel + g0
            eid_c = eid_r + _BLK
            rx = plsc.load_gather(rows_v, [eid_r, comp[0]])
            ry = plsc.load_gather(rows_v, [eid_r, comp[1]])
            rz = plsc.load_gather(rows_v, [eid_r, comp[2]])
            cx = plsc.load_gather(rows_v, [eid_c, comp[0]])
            cy = plsc.load_gather(rows_v, [eid_c, comp[1]])
            cz = plsc.load_gather(rows_v, [eid_c, comp[2]])
            osl = pl.ds(e0, _LANES)
            dx = cx - rx
            dy = cy - ry
            dz = cz - rz
            s = dx * dx + dy * dy + dz * dz
            s_bits = lax.bitcast_convert_type(s, jnp.int32)
            y = lax.bitcast_convert_type(magic - (s_bits >> 1), jnp.float32)
            xh = s * half
            y = y * (threehalf - xh * y * y)
            y = y * (threehalf - xh * y * y)
            y = y * (threehalf - xh * y * y)
            h = y * half
            ox_v[osl] = dx * h + half
            oy_v[osl] = dy * h + half
            oz_v[osl] = dz * h + half
            return _

        lax.fori_loop(0, n_vec, vec_body, None)
        pltpu.sync_copy(ox_v, ox_hbm.at[pl.ds(off, chunk)])
        pltpu.sync_copy(oy_v, oy_hbm.at[pl.ds(off, chunk)])
        pltpu.sync_copy(oz_v, oz_hbm.at[pl.ds(off, chunk)])
        return _

    lax.fori_loop(0, n_mine, chunk_body, None)


@functools.cache
def _build(n_edges: int):
    chunk = _pick_chunk(n_edges)
    n_chunks = n_edges // chunk
    mesh = plsc.VectorSubcoreMesh(core_axis_name="c", subcore_axis_name="s",
                                  num_cores=_NUM_CORES,
                                  num_subcores=_NUM_SUBCORES)
    comp = jax.ShapeDtypeStruct((n_edges,), jnp.float32)
    return pl.kernel(
        functools.partial(_sc_body, chunk=chunk, n_chunks=n_chunks),
        out_type=(comp, comp, comp),
        mesh=mesh,
        scratch_types=[
            pltpu.VMEM((chunk * 4,), jnp.int32),
            pltpu.VMEM((chunk * 4, 8), jnp.float32),
            pltpu.VMEM((chunk,), jnp.float32),
            pltpu.VMEM((chunk,), jnp.float32),
            pltpu.VMEM((chunk,), jnp.float32),
            pltpu.SemaphoreType.DMA((2,)),
        ],
        compiler_params=pltpu.CompilerParams(needs_layout_passes=False,
                                             use_tc_tiling_on_sc=False),
    )


def kernel(pos, edge_index, edge_weight):
    n_edges = edge_weight.shape[0]
    pos4 = jnp.concatenate(
        [pos.astype(jnp.float32),
         jnp.zeros((pos.shape[0], 5), jnp.float32)], axis=1)
    ei_blk = (edge_index.astype(jnp.int32)
              .reshape(2, n_edges // _BLK, _BLK)
              .transpose(1, 0, 2)
              .reshape(2 * n_edges))
    ox, oy, oz = _build(n_edges)(pos4, ei_blk)
    return jnp.stack([ox, oy, oz, edge_weight.astype(jnp.float32)], axis=1)


# speedup vs baseline: 8.2919x; 1.0322x over previous
import functools

import jax
import jax.numpy as jnp
from jax import lax
from jax.experimental import pallas as pl
from jax.experimental.pallas import tpu as pltpu
from jax.experimental.pallas import tpu_sc as plsc

_NUM_CORES = 2
_NUM_SUBCORES = 16
_NUM_WORKERS = _NUM_CORES * _NUM_SUBCORES
_LANES = 16
_BLK = 128


def _pick_chunk(n_edges: int) -> int:
    for c in range(2560, _BLK - 1, -_BLK):
        if n_edges % c == 0:
            return c
    return _BLK


def _sc_body(pos4_hbm, ei_hbm, ox_hbm, oy_hbm, oz_hbm,
             idx_v, rows_v, ox_v, oy_v, oz_v, sem,
             *, chunk: int, n_chunks: int):
    wid = lax.axis_index("s") * _NUM_CORES + lax.axis_index("c")
    n_mine = (n_chunks - wid + _NUM_WORKERS - 1) // _NUM_WORKERS
    n_vec = chunk // _LANES
    grp_per_blk = _BLK // _LANES

    lane_iota = lax.iota(jnp.int32, _LANES)
    comp = [jnp.full((_LANES,), j, jnp.int32) for j in range(3)]
    half = jnp.float32(0.5)
    threehalf = jnp.float32(1.5)
    magic = jnp.int32(0x5F3759DF)

    def start_fetch(j):
        sel = (j % 2) * chunk * 2
        off = (wid + j * _NUM_WORKERS) * chunk
        idx_half = idx_v.at[pl.ds(sel, chunk * 2)]
        pltpu.sync_copy(ei_hbm.at[pl.ds(off * 2, chunk * 2)], idx_half)
        pltpu.async_copy(pos4_hbm.at[idx_half],
                         rows_v.at[pl.ds(sel, chunk * 2), :],
                         sem.at[j % 2])

    start_fetch(0)

    def chunk_body(j, _):
        k = wid + j * _NUM_WORKERS
        off = k * chunk
        sel = (j % 2) * chunk * 2

        @pl.when(j + 1 < n_mine)
        def _prefetch():
            start_fetch(j + 1)

        pltpu.make_async_copy(pos4_hbm.at[idx_v.at[pl.ds(sel, chunk * 2)]],
                              rows_v.at[pl.ds(sel, chunk * 2), :],
                              sem.at[j % 2]).wait()

        def blk_body(b, _):
            r0 = sel + b * (2 * _BLK)
            e0b = b * _BLK
            for t in range(grp_per_blk):
                eid_r = lane_iota + (r0 + t * _LANES)
                eid_c = eid_r + _BLK
                rx = plsc.load_gather(rows_v, [eid_r, comp[0]])
                ry = plsc.load_gather(rows_v, [eid_r, comp[1]])
                rz = plsc.load_gather(rows_v, [eid_r, comp[2]])
                cx = plsc.load_gather(rows_v, [eid_c, comp[0]])
                cy = plsc.load_gather(rows_v, [eid_c, comp[1]])
                cz = plsc.load_gather(rows_v, [eid_c, comp[2]])
                osl = pl.ds(e0b + t * _LANES, _LANES)
                dx = cx - rx
                dy = cy - ry
                dz = cz - rz
                s = dx * dx + dy * dy + dz * dz
                s_bits = lax.bitcast_convert_type(s, jnp.int32)
                y = lax.bitcast_convert_type(magic - (s_bits >> 1),
                                             jnp.float32)
                xh = s * half
                y = y * (threehalf - xh * y * y)
                y = y * (threehalf - xh * y * y)
                h = y * half
                ox_v[osl] = dx * h + half
                oy_v[osl] = dy * h + half
                oz_v[osl] = dz * h + half
            return _

        lax.fori_loop(0, chunk // _BLK, blk_body, None)
        pltpu.sync_copy(ox_v, ox_hbm.at[pl.ds(off, chunk)])
        pltpu.sync_copy(oy_v, oy_hbm.at[pl.ds(off, chunk)])
        pltpu.sync_copy(oz_v, oz_hbm.at[pl.ds(off, chunk)])
        return _

    lax.fori_loop(0, n_mine, chunk_body, None)


@functools.cache
def _build(n_edges: int):
    chunk = _pick_chunk(n_edges)
    n_chunks = n_edges // chunk
    mesh = plsc.VectorSubcoreMesh(core_axis_name="c", subcore_axis_name="s",
                                  num_cores=_NUM_CORES,
                                  num_subcores=_NUM_SUBCORES)
    comp = jax.ShapeDtypeStruct((n_edges,), jnp.float32)
    return pl.kernel(
        functools.partial(_sc_body, chunk=chunk, n_chunks=n_chunks),
        out_type=(comp, comp, comp),
        mesh=mesh,
        scratch_types=[
            pltpu.VMEM((chunk * 4,), jnp.int32),
            pltpu.VMEM((chunk * 4, 8), jnp.float32),
            pltpu.VMEM((chunk,), jnp.float32),
            pltpu.VMEM((chunk,), jnp.float32),
            pltpu.VMEM((chunk,), jnp.float32),
            pltpu.SemaphoreType.DMA((2,)),
        ],
        compiler_params=pltpu.CompilerParams(needs_layout_passes=False,
                                             use_tc_tiling_on_sc=False),
    )


def kernel(pos, edge_index, edge_weight):
    n_edges = edge_weight.shape[0]
    pos4 = jnp.concatenate(
        [pos.astype(jnp.float32),
         jnp.zeros((pos.shape[0], 5), jnp.float32)], axis=1)
    ei_blk = (edge_index.astype(jnp.int32)
              .reshape(2, n_edges // _BLK, _BLK)
              .transpose(1, 0, 2)
              .reshape(2 * n_edges))
    ox, oy, oz = _build(n_edges)(pos4, ei_blk)
    return jnp.stack([ox, oy, oz, edge_weight.astype(jnp.float32)], axis=1)


# speedup vs baseline: 8.5612x; 1.0325x over previous
import functools

import jax
import jax.numpy as jnp
from jax import lax
from jax.experimental import pallas as pl
from jax.experimental.pallas import tpu as pltpu
from jax.experimental.pallas import tpu_sc as plsc

_NUM_CORES = 2
_NUM_SUBCORES = 16
_NUM_WORKERS = _NUM_CORES * _NUM_SUBCORES
_LANES = 16
_BLK = 128


def _pick_chunk(n_edges: int) -> int:
    for c in range(2560, _BLK - 1, -_BLK):
        if n_edges % c == 0:
            return c
    return _BLK


def _sc_body(pos4_hbm, ei_hbm, ox_hbm, oy_hbm, oz_hbm,
             idx_v, rows_v, ox_v, oy_v, oz_v,
             sem_idx, sem_row, sem_out,
             *, chunk: int, n_chunks: int):
    wid = lax.axis_index("s") * _NUM_CORES + lax.axis_index("c")
    n_mine = (n_chunks - wid + _NUM_WORKERS - 1) // _NUM_WORKERS
    grp_per_blk = _BLK // _LANES

    lane_iota = lax.iota(jnp.int32, _LANES)
    comp = [jnp.full((_LANES,), j, jnp.int32) for j in range(3)]
    half = jnp.float32(0.5)
    threehalf = jnp.float32(1.5)
    magic = jnp.int32(0x5F3759DF)

    def idx_half(j):
        return idx_v.at[pl.ds((j % 2) * chunk * 2, chunk * 2)]

    def rows_half(j):
        return rows_v.at[pl.ds((j % 2) * chunk * 2, chunk * 2), :]

    def ei_src(j):
        off = (wid + j * _NUM_WORKERS) * chunk
        return ei_hbm.at[pl.ds(off * 2, chunk * 2)]

    def out_parts(j):
        o = pl.ds((j % 2) * chunk, chunk)
        off = (wid + j * _NUM_WORKERS) * chunk
        h = pl.ds(off, chunk)
        return ((ox_v.at[o], ox_hbm.at[h]), (oy_v.at[o], oy_hbm.at[h]),
                (oz_v.at[o], oz_hbm.at[h]))

    def idx_copy(j):
        return pltpu.make_async_copy(ei_src(j), idx_half(j),
                                     sem_idx.at[j % 2])

    def row_copy(j):
        return pltpu.make_async_copy(pos4_hbm.at[idx_half(j)], rows_half(j),
                                     sem_row.at[j % 2])

    idx_copy(0).start()
    idx_copy(1).start()
    idx_copy(0).wait()
    row_copy(0).start()

    def chunk_body(j, _):
        sel = (j % 2) * chunk * 2

        @pl.when(j + 1 < n_mine)
        def _fire_next_gather():
            idx_copy(j + 1).wait()
            row_copy(j + 1).start()

        row_copy(j).wait()

        @pl.when(j + 2 < n_mine)
        def _fire_next_idx():
            idx_copy(j + 2).start()

        @pl.when(j >= 2)
        def _drain_out():
            for src, dst in out_parts(j - 2):
                pltpu.make_async_copy(src, dst, sem_out.at[j % 2]).wait()

        osel = (j % 2) * chunk

        def blk_body(b, _):
            r0 = sel + b * (2 * _BLK)
            e0b = osel + b * _BLK
            for t in range(grp_per_blk):
                eid_r = lane_iota + (r0 + t * _LANES)
                eid_c = eid_r + _BLK
                rx = plsc.load_gather(rows_v, [eid_r, comp[0]])
                ry = plsc.load_gather(rows_v, [eid_r, comp[1]])
                rz = plsc.load_gather(rows_v, [eid_r, comp[2]])
                cx = plsc.load_gather(rows_v, [eid_c, comp[0]])
                cy = plsc.load_gather(rows_v, [eid_c, comp[1]])
                cz = plsc.load_gather(rows_v, [eid_c, comp[2]])
                osl = pl.ds(e0b + t * _LANES, _LANES)
                dx = cx - rx
                dy = cy - ry
                dz = cz - rz
                s = dx * dx + dy * dy + dz * dz
                s_bits = lax.bitcast_convert_type(s, jnp.int32)
                y = lax.bitcast_convert_type(magic - (s_bits >> 1),
                                             jnp.float32)
                xh = s * half
                y = y * (threehalf - xh * y * y)
                y = y * (threehalf - xh * y * y)
                h = y * half
                ox_v[osl] = dx * h + half
                oy_v[osl] = dy * h + half
                oz_v[osl] = dz * h + half
            return _

        lax.fori_loop(0, chunk // _BLK, blk_body, None)
        for src, dst in out_parts(j):
            pltpu.make_async_copy(src, dst, sem_out.at[j % 2]).start()
        return _

    lax.fori_loop(0, n_mine, chunk_body, None)

    def drain_body(j, _):
        @pl.when(j >= lax.max(n_mine - 2, 0))
        def _():
            for src, dst in out_parts(j):
                pltpu.make_async_copy(src, dst, sem_out.at[j % 2]).wait()
        return _

    lax.fori_loop(lax.max(n_mine - 2, 0), n_mine, drain_body, None)


@functools.cache
def _build(n_edges: int):
    chunk = _pick_chunk(n_edges)
    n_chunks = n_edges // chunk
    mesh = plsc.VectorSubcoreMesh(core_axis_name="c", subcore_axis_name="s",
                                  num_cores=_NUM_CORES,
                                  num_subcores=_NUM_SUBCORES)
    comp = jax.ShapeDtypeStruct((n_edges,), jnp.float32)
    return pl.kernel(
        functools.partial(_sc_body, chunk=chunk, n_chunks=n_chunks),
        out_type=(comp, comp, comp),
        mesh=mesh,
        scratch_types=[
            pltpu.VMEM((chunk * 4,), jnp.int32),
            pltpu.VMEM((chunk * 4, 8), jnp.float32),
            pltpu.VMEM((chunk * 2,), jnp.float32),
            pltpu.VMEM((chunk * 2,), jnp.float32),
            pltpu.VMEM((chunk * 2,), jnp.float32),
            pltpu.SemaphoreType.DMA((2,)),
            pltpu.SemaphoreType.DMA((2,)),
            pltpu.SemaphoreType.DMA((2,)),
        ],
        compiler_params=pltpu.CompilerParams(needs_layout_passes=False,
                                             use_tc_tiling_on_sc=False),
    )


def kernel(pos, edge_index, edge_weight):
    n_edges = edge_weight.shape[0]
    pos4 = jnp.concatenate(
        [pos.astype(jnp.float32),
         jnp.zeros((pos.shape[0], 5), jnp.float32)], axis=1)
    ei_blk = (edge_index.astype(jnp.int32)
              .reshape(2, n_edges // _BLK, _BLK)
              .transpose(1, 0, 2)
              .reshape(2 * n_edges))
    ox, oy, oz = _build(n_edges)(pos4, ei_blk)
    return jnp.stack([ox, oy, oz, edge_weight.astype(jnp.float32)], axis=1)
